# baseline (speedup 1.0000x reference)
import jax
import jax.numpy as jnp
from jax import lax
from jax.experimental import pallas as pl
from jax.experimental.pallas import tpu as pltpu
from jax.experimental.pallas import tpu_sc as plsc

NU = 10000
NI = 10000
D = 128
E = 320000

NC = 2
NS = 16
LANES = 16
NW = NC * NS
EPT = E // NW
K = 80
NCHUNK = EPT // K
ACCR = 10240
RPT = ACCR // NS
RW = 128
NRW = RPT // RW

_f32 = jnp.float32


_BLK = 1000
_GRID = NU // _BLK


def _row_block(i):
    return (i, 0)


def _bcast_block(i):
    return (0, 0)


def _part_block(i):
    return (0, i, 0)


def _att_mix(e1, e2, e3, q):
    w1 = jnp.sum(e1 * q, axis=1, keepdims=True)
    w2 = jnp.sum(e2 * q, axis=1, keepdims=True)
    w3 = jnp.sum(e3 * q, axis=1, keepdims=True)
    m = jnp.maximum(jnp.maximum(w1, w2), w3)
    x1 = jnp.exp(w1 - m)
    x2 = jnp.exp(w2 - m)
    x3 = jnp.exp(w3 - m)
    s = x1 + x2 + x3
    return (e1 * x1 + e2 * x2 + e3 * x3) / s


def _l2n(x):
    nrm = jnp.sqrt(jnp.sum(x * x, axis=1, keepdims=True))
    return x / jnp.maximum(nrm, 1e-12)


def _tc_init_body(x_ref, w1_ref, b1_ref, w2_ref, b2_ref, w3_ref, b3_ref,
                  w4_ref, b4_ref, am_ref, av_ref,
                  c1_o, c2_o, c3_o, sp_o, mix_o):
    x = x_ref[...]

    def gate(w_ref, b_ref):
        t = jnp.dot(x, w_ref[...], preferred_element_type=_f32) + b_ref[...]
        return x * jax.nn.sigmoid(t)

    c1 = gate(w1_ref, b1_ref)
    c2 = gate(w2_ref, b2_ref)
    c3 = gate(w3_ref, b3_ref)
    sp = gate(w4_ref, b4_ref)
    q = jnp.dot(am_ref[...], av_ref[...].T, preferred_element_type=_f32).T
    c1_o[...] = c1
    c2_o[...] = c2
    c3_o[...] = c3
    sp_o[...] = sp
    mix_o[...] = _att_mix(c1, c2, c3, q) + 0.5 * sp


def _tc_init(user_emb, W1, b1, W2, b2, W3, b3, W4, b4, am, av):
    row = pl.BlockSpec((_BLK, D), _row_block)
    wspec = pl.BlockSpec((D, D), _bcast_block)
    bspec = pl.BlockSpec((1, D), _bcast_block)
    out = jax.ShapeDtypeStruct((NU, D), _f32)
    return pl.pallas_call(
        _tc_init_body,
        grid=(_GRID,),
        in_specs=[row, wspec, bspec, wspec, bspec, wspec, bspec, wspec,
                  bspec, wspec, bspec],
        out_specs=[row] * 5,
        out_shape=[out] * 5,
    )(user_emb, W1, b1.reshape(1, D), W2, b2.reshape(1, D),
      W3, b3.reshape(1, D), W4, b4.reshape(1, D), am, av)


def _tc_update_body(p1_ref, p2_ref, p3_ref, pit_ref, psp_ref,
                    a1_ref, a2_ref, a3_ref, asp_ref, ai_ref,
                    am_ref, av_ref,
                    c1_o, c2_o, c3_o, sp_o, it_o,
                    a1_o, a2_o, a3_o, asp_o, ai_o, mix_o):
    c1 = p1_ref[0] + p1_ref[1]
    c2 = p2_ref[0] + p2_ref[1]
    c3 = p3_ref[0] + p3_ref[1]
    it = pit_ref[0] + pit_ref[1]
    sp = psp_ref[0] + psp_ref[1]
    c1_o[...] = c1
    c2_o[...] = c2
    c3_o[...] = c3
    sp_o[...] = sp
    it_o[...] = it
    a1_o[...] = a1_ref[...] + _l2n(c1)
    a2_o[...] = a2_ref[...] + _l2n(c2)
    a3_o[...] = a3_ref[...] + _l2n(c3)
    asp_o[...] = asp_ref[...] + _l2n(sp)
    ai_o[...] = ai_ref[...] + _l2n(it)
    q = jnp.dot(am_ref[...], av_ref[...].T, preferred_element_type=_f32).T
    mix_o[...] = _att_mix(c1, c2, c3, q) + 0.5 * sp


def _tc_update(P1, P2, P3, Pit, Psp, A1, A2, A3, Asp, Ai, am, av):
    part = pl.BlockSpec((NC, _BLK, D), _part_block)
    row = pl.BlockSpec((_BLK, D), _row_block)
    wspec = pl.BlockSpec((D, D), _bcast_block)
    bspec = pl.BlockSpec((1, D), _bcast_block)
    out = jax.ShapeDtypeStruct((NU, D), _f32)
    return pl.pallas_call(
        _tc_update_body,
        grid=(_GRID,),
        in_specs=[part] * 5 + [row] * 5 + [wspec, bspec],
        out_specs=[row] * 11,
        out_shape=[out] * 11,
    )(P1, P2, P3, Pit, Psp, A1, A2, A3, Asp, Ai, am, av)


def _tc_final_body(p1_ref, p2_ref, p3_ref, pit_ref, psp_ref,
                   a1_ref, a2_ref, a3_ref, asp_ref, ai_ref,
                   am_ref, av_ref, user_o, item_o):
    c1 = p1_ref[0] + p1_ref[1]
    c2 = p2_ref[0] + p2_ref[1]
    c3 = p3_ref[0] + p3_ref[1]
    it = pit_ref[0] + pit_ref[1]
    sp = psp_ref[0] + psp_ref[1]
    u1 = a1_ref[...] + _l2n(c1)
    u2 = a2_ref[...] + _l2n(c2)
    u3 = a3_ref[...] + _l2n(c3)
    usp = asp_ref[...] + _l2n(sp)
    item_o[...] = ai_ref[...] + _l2n(it)
    q = jnp.dot(am_ref[...], av_ref[...].T, preferred_element_type=_f32).T
    user_o[...] = _att_mix(u1, u2, u3, q) + 0.5 * usp


def _tc_final(P1, P2, P3, Pit, Psp, A1, A2, A3, Asp, Ai, am, av):
    part = pl.BlockSpec((NC, _BLK, D), _part_block)
    row = pl.BlockSpec((_BLK, D), _row_block)
    wspec = pl.BlockSpec((D, D), _bcast_block)
    bspec = pl.BlockSpec((1, D), _bcast_block)
    out = jax.ShapeDtypeStruct((NU, D), _f32)
    return pl.pallas_call(
        _tc_final_body,
        grid=(_GRID,),
        in_specs=[part] * 5 + [row] * 5 + [wspec, bspec],
        out_specs=[row] * 2,
        out_shape=[out] * 2,
    )(P1, P2, P3, Pit, Psp, A1, A2, A3, Asp, Ai, am, av)


def _sc_layer_body(c1_h, c2_h, c3_h, mix_h, it_h,
                   rs_h, cs_h, vs_h, rj_h, cj_h, vj_h,
                   rp_h, cp_h, vp_h, rr_h, cr_h, vr_h,
                   o1_h, o2_h, o3_h, oit_h, osp_h,
                   acc, idx_r, idx_c, val_v, gx, zbuf, wbuf, sem):
    cid = lax.axis_index("c")
    sid = lax.axis_index("s")
    base = (cid * NS + sid) * EPT
    r0 = sid * RPT

    def zrow(r, _):
        for j in range(D // LANES):
            zbuf[r, pl.ds(j * LANES, LANES)] = jnp.zeros((LANES,), _f32)
        return 0

    lax.fori_loop(0, RW, zrow, 0)

    jobs = [
        (rs_h, cs_h, vs_h, c1_h, o1_h),
        (rj_h, cj_h, vj_h, c2_h, o2_h),
        (rp_h, cp_h, vp_h, c3_h, o3_h),
        (cr_h, rr_h, vr_h, mix_h, oit_h),
        (rr_h, cr_h, vr_h, it_h, osp_h),
    ]

    for rows_h, cols_h, vals_h, x_h, out_h in jobs:
        for m in range(NRW):
            pltpu.sync_copy(zbuf, acc.at[pl.ds(r0 + m * RW, RW), :])
        plsc.subcore_barrier()

        def chunk(t, _):
            off = base + t * K
            pltpu.sync_copy(rows_h.at[pl.ds(off, K)], idx_r)
            pltpu.sync_copy(cols_h.at[pl.ds(off, K)], idx_c)
            pltpu.sync_copy(vals_h.at[pl.ds(off, K)], val_v)
            pltpu.async_copy(x_h.at[idx_c], gx, sem).wait()

            def scale16(g, _):
                vv = val_v[pl.ds(g * LANES, LANES)]
                for l in range(LANES):
                    s = vv[l]
                    e = g * LANES + l
                    for j in range(D // LANES):
                        sl = pl.ds(j * LANES, LANES)
                        gx[e, sl] = gx[e, sl] * s
                return 0

            lax.fori_loop(0, K // LANES, scale16, 0)
            pltpu.sync_copy(gx, acc.at[idx_r], add=True)
            return 0

        lax.fori_loop(0, NCHUNK, chunk, 0)
        plsc.subcore_barrier()

        for m in range(NRW):
            rr0 = r0 + m * RW
            pltpu.sync_copy(acc.at[pl.ds(rr0, RW), :], wbuf)
            pltpu.sync_copy(wbuf, out_h.at[cid, pl.ds(rr0, RW), :])


def _sc_layer(c1, c2, c3, mix, it,
              rows_s, cols_s, vals_s, rows_j, cols_j, vals_j,
              rows_p, cols_p, vals_p, rows_r, cols_r, vals_r):
    mesh = plsc.VectorSubcoreMesh(core_axis_name="c", subcore_axis_name="s",
                                  num_cores=NC, num_subcores=NS)
    part = jax.ShapeDtypeStruct((NC, ACCR, D), _f32)
    fn = pl.kernel(
        _sc_layer_body,
        out_type=[part] * 5,
        mesh=mesh,
        scratch_types=[
            pltpu.VMEM_SHARED((ACCR, D), _f32),
            pltpu.VMEM((K,), jnp.int32),
            pltpu.VMEM((K,), jnp.int32),
            pltpu.VMEM((K,), _f32),
            pltpu.VMEM((K, D), _f32),
            pltpu.VMEM((RW, D), _f32),
            pltpu.VMEM((RW, D), _f32),
            pltpu.SemaphoreType.DMA,
        ],
    )
    return fn(c1, c2, c3, mix, it,
              rows_s, cols_s, vals_s, rows_j, cols_j, vals_j,
              rows_p, cols_p, vals_p, rows_r, cols_r, vals_r)


def kernel(user_emb, item_emb, rows_s, cols_s, vals_s, rows_j, cols_j,
           vals_j, rows_p, cols_p, vals_p, rows_r, cols_r, vals_r,
           W1, b1, W2, b2, W3, b3, W4, b4, att_mat, att_vec):
    c1, c2, c3, sp, mix = _tc_init(user_emb, W1, b1, W2, b2, W3, b3, W4, b4,
                                   att_mat, att_vec)
    it = item_emb
    A1, A2, A3, Asp, Ai = c1, c2, c3, sp, it

    edges = (rows_s, cols_s, vals_s, rows_j, cols_j, vals_j,
             rows_p, cols_p, vals_p, rows_r, cols_r, vals_r)

    P1, P2, P3, Pit, Psp = _sc_layer(c1, c2, c3, mix, it, *edges)
    (c1, c2, c3, sp, it, A1, A2, A3, Asp, Ai, mix) = _tc_update(
        P1, P2, P3, Pit, Psp, A1, A2, A3, Asp, Ai, att_mat, att_vec)

    P1, P2, P3, Pit, Psp = _sc_layer(c1, c2, c3, mix, it, *edges)
    user_all, item_all = _tc_final(
        P1, P2, P3, Pit, Psp, A1, A2, A3, Asp, Ai, att_mat, att_vec)
    return (user_all, item_all)
